# Initial kernel scaffold; baseline (speedup 1.0000x reference)
import jax
import jax.numpy as jnp
from jax.experimental import pallas as pl


def kernel(x, edge_index, edge_type, edge_table, Win, b_in, Ws1, as1, ad1, We1, ae1, b1, Ws2, as2, ad2, We2, ae2, b2, Wout, bout):
    raise NotImplementedError("write your pallas kernel here")


# speedup vs baseline: 23.9424x; 23.9424x over previous
import functools
import jax
import jax.numpy as jnp
from jax import lax
from jax.experimental import pallas as pl
from jax.experimental.pallas import tpu as pltpu
from jax.experimental.pallas import tpu_sc as plsc

N = 10000
D = 128
E = 320000
NC, NS, LANES = 2, 16, 16
NW = NC * NS
C = 128
CHUNKS = 79
EW = CHUNKS * C
E_PAD = EW * NW
N_PAD = 10240
RPT = N_PAD // NS
EPS = 1e-16


def _bcast_lane(v, lane):
    idx = jnp.full((LANES,), lane, dtype=jnp.int32)
    dn = lax.GatherDimensionNumbers(
        offset_dims=(), collapsed_slice_dims=(0,), start_index_map=(0,))
    return lax.gather(v, idx[:, None], dn, (1,),
                      mode=lax.GatherScatterMode.PROMISE_IN_BOUNDS)


def _gat_sc_body(xs_hbm, asrc_hbm, adst_hbm, ta_hbm, src_hbm, dst_hbm, et_hbm,
                 agg_out, den_out,
                 asrc_v, adst_v, ta_v, src_v, dst_v, et_v, ex_v, rows_v,
                 zden_v, agg_s, den_s, sem):
    c = lax.axis_index("c")
    s = lax.axis_index("s")
    w = s * NC + c
    zero = jnp.zeros((LANES,), jnp.float32)

    pltpu.sync_copy(asrc_hbm, asrc_v)
    pltpu.sync_copy(adst_hbm, adst_v.at[pl.ds(0, N)])
    pltpu.sync_copy(ta_hbm, ta_v)
    adst_v[pl.ds(N, LANES)] = zero
    adst_v[pl.ds(N + LANES, LANES)] = zero

    @pl.loop(0, C)
    def _zr(r):
        for cv in range(8):
            rows_v[r, pl.ds(cv * LANES, LANES)] = zero

    @pl.loop(0, RPT // LANES)
    def _zd(i):
        zden_v[pl.ds(i * LANES, LANES)] = zero

    row0 = s * RPT
    for k in range(RPT // C):
        pltpu.sync_copy(rows_v, agg_s.at[pl.ds(row0 + k * C, C)])
    pltpu.sync_copy(zden_v, den_s.at[pl.ds(row0, RPT)])
    plsc.subcore_barrier()

    @pl.loop(0, CHUNKS)
    def _chunk(k):
        base = w * EW + k * C
        pltpu.sync_copy(src_hbm.at[pl.ds(base, C)], src_v)
        pltpu.sync_copy(dst_hbm.at[pl.ds(base, C)], dst_v)
        pltpu.sync_copy(et_hbm.at[pl.ds(base, C)], et_v)
        pltpu.async_copy(xs_hbm.at[src_v], rows_v, sem).wait()
        for j in range(C // LANES):
            sl = pl.ds(j * LANES, LANES)
            a = (plsc.load_gather(asrc_v, [src_v[sl]])
                 + plsc.load_gather(adst_v, [dst_v[sl]])
                 + plsc.load_gather(ta_v, [et_v[sl]]))
            a = jnp.maximum(a, 0.2 * a)
            ex_v[sl] = jnp.exp(a)

        @pl.loop(0, C // LANES)
        def _scale(j):
            exr = ex_v[pl.ds(j * LANES, LANES)]
            for lane in range(LANES):
                b = _bcast_lane(exr, lane)
                e = j * LANES + lane
                for cv in range(8):
                    sl2 = pl.ds(cv * LANES, LANES)
                    rows_v[e, sl2] = rows_v[e, sl2] * b

        pltpu.sync_copy(ex_v, den_s.at[dst_v], add=True)
        pltpu.sync_copy(rows_v, agg_s.at[dst_v], add=True)

    plsc.subcore_barrier()
    pltpu.sync_copy(agg_s.at[pl.ds(row0, RPT)],
                    agg_out.at[c, pl.ds(row0, RPT)])
    pltpu.sync_copy(den_s.at[pl.ds(row0, RPT)],
                    den_out.at[c, pl.ds(row0, RPT)])


def _gat_sc_layer(xs, asrc, adst, ta, src, dst, et):
    mesh = plsc.VectorSubcoreMesh(core_axis_name="c", subcore_axis_name="s",
                                  num_cores=NC, num_subcores=NS)
    f = pl.kernel(
        _gat_sc_body,
        out_type=(jax.ShapeDtypeStruct((NC, N_PAD, D), jnp.float32),
                  jax.ShapeDtypeStruct((NC, N_PAD), jnp.float32)),
        mesh=mesh,
        scratch_types=[
            pltpu.VMEM((N,), jnp.float32),
            pltpu.VMEM((N + 2 * LANES,), jnp.float32),
            pltpu.VMEM((128,), jnp.float32),
            pltpu.VMEM((C,), jnp.int32),
            pltpu.VMEM((C,), jnp.int32),
            pltpu.VMEM((C,), jnp.int32),
            pltpu.VMEM((C,), jnp.float32),
            pltpu.VMEM((C, D), jnp.float32),
            pltpu.VMEM((RPT,), jnp.float32),
            pltpu.VMEM_SHARED((N_PAD, D), jnp.float32),
            pltpu.VMEM_SHARED((N_PAD,), jnp.float32),
            pltpu.SemaphoreType.DMA,
        ],
        compiler_params=pltpu.CompilerParams(needs_layout_passes=False),
        name="gat_edge_aggregate",
    )
    return f(xs, asrc, adst, ta, src, dst, et)


ROWS_BLK = 400
GRID = N // ROWS_BLK


def _stage1_body(x_ref, win_ref, bin_ref, ws_ref, as_ref, ad_ref,
                 etab_ref, we_ref, ae_ref,
                 xs_ref, asrc_ref, adst_ref, ta_ref):
    h = jnp.dot(x_ref[...], win_ref[...],
                preferred_element_type=jnp.float32) + bin_ref[...]
    xs = jnp.dot(h, ws_ref[...], preferred_element_type=jnp.float32)
    xs_ref[...] = xs
    asrc_ref[...] = jnp.dot(xs, as_ref[...], preferred_element_type=jnp.float32)
    adst_ref[...] = jnp.dot(xs, ad_ref[...], preferred_element_type=jnp.float32)
    ee = jnp.dot(etab_ref[...], we_ref[...], preferred_element_type=jnp.float32)
    ta_ref[...] = jnp.dot(ee, ae_ref[...], preferred_element_type=jnp.float32)


def _stage2_body(p0_ref, p1_ref, d0_ref, d1_ref, bprev_ref, ws_ref, as_ref,
                 ad_ref, etab_ref, we_ref, ae_ref,
                 xs_ref, asrc_ref, adst_ref, ta_ref):
    agg = p0_ref[...] + p1_ref[...]
    den = d0_ref[...] + d1_ref[...] + EPS
    h = jnp.maximum(agg / den + bprev_ref[...], 0.0)
    xs = jnp.dot(h, ws_ref[...], preferred_element_type=jnp.float32)
    xs_ref[...] = xs
    asrc_ref[...] = jnp.dot(xs, as_ref[...], preferred_element_type=jnp.float32)
    adst_ref[...] = jnp.dot(xs, ad_ref[...], preferred_element_type=jnp.float32)
    ee = jnp.dot(etab_ref[...], we_ref[...], preferred_element_type=jnp.float32)
    ta_ref[...] = jnp.dot(ee, ae_ref[...], preferred_element_type=jnp.float32)


def _stage3_body(p0_ref, p1_ref, d0_ref, d1_ref, b2_ref, wout_ref, bout_ref,
                 out_ref):
    agg = p0_ref[...] + p1_ref[...]
    den = d0_ref[...] + d1_ref[...] + EPS
    h = jnp.maximum(agg / den + b2_ref[...], 0.0)
    out_ref[...] = jnp.dot(h, wout_ref[...],
                           preferred_element_type=jnp.float32) + bout_ref[...]


def _row_spec(blk):
    return pl.BlockSpec(blk, lambda i: (0,) * len(blk))


def _blk_spec(blk):
    return pl.BlockSpec(blk, lambda i: (i,) + (0,) * (len(blk) - 1))


def _tc_stage1(x, win, b_in, ws, a_s, a_d, etab, we, a_e):
    return pl.pallas_call(
        _stage1_body,
        grid=(GRID,),
        in_specs=[
            _blk_spec((ROWS_BLK, D)),
            _row_spec((D, D)), _row_spec((1, D)), _row_spec((D, D)),
            _row_spec((D, 1)), _row_spec((D, 1)),
            _row_spec((128, 16)), _row_spec((16, D)), _row_spec((D, 1)),
        ],
        out_specs=[
            _blk_spec((ROWS_BLK, D)), _blk_spec((ROWS_BLK, 1)),
            _blk_spec((ROWS_BLK, 1)), _row_spec((128, 1)),
        ],
        out_shape=[
            jax.ShapeDtypeStruct((N, D), jnp.float32),
            jax.ShapeDtypeStruct((N, 1), jnp.float32),
            jax.ShapeDtypeStruct((N, 1), jnp.float32),
            jax.ShapeDtypeStruct((128, 1), jnp.float32),
        ],
    )(x, win, b_in, ws, a_s, a_d, etab, we, a_e)


def _tc_stage2(p0, p1, d0, d1, bprev, ws, a_s, a_d, etab, we, a_e):
    return pl.pallas_call(
        _stage2_body,
        grid=(GRID,),
        in_specs=[
            _blk_spec((ROWS_BLK, D)), _blk_spec((ROWS_BLK, D)),
            _blk_spec((ROWS_BLK, 1)), _blk_spec((ROWS_BLK, 1)),
            _row_spec((1, D)), _row_spec((D, D)),
            _row_spec((D, 1)), _row_spec((D, 1)),
            _row_spec((128, 16)), _row_spec((16, D)), _row_spec((D, 1)),
        ],
        out_specs=[
            _blk_spec((ROWS_BLK, D)), _blk_spec((ROWS_BLK, 1)),
            _blk_spec((ROWS_BLK, 1)), _row_spec((128, 1)),
        ],
        out_shape=[
            jax.ShapeDtypeStruct((N, D), jnp.float32),
            jax.ShapeDtypeStruct((N, 1), jnp.float32),
            jax.ShapeDtypeStruct((N, 1), jnp.float32),
            jax.ShapeDtypeStruct((128, 1), jnp.float32),
        ],
    )(p0, p1, d0, d1, bprev, ws, a_s, a_d, etab, we, a_e)


def _tc_stage3(p0, p1, d0, d1, b2, wout, bout):
    return pl.pallas_call(
        _stage3_body,
        grid=(GRID,),
        in_specs=[
            _blk_spec((ROWS_BLK, D)), _blk_spec((ROWS_BLK, D)),
            _blk_spec((ROWS_BLK, 1)), _blk_spec((ROWS_BLK, 1)),
            _row_spec((1, D)), _row_spec((D, 1)), _row_spec((1, 1)),
        ],
        out_specs=_blk_spec((ROWS_BLK, 1)),
        out_shape=jax.ShapeDtypeStruct((N, 1), jnp.float32),
    )(p0, p1, d0, d1, b2, wout, bout)


@jax.jit
def kernel(x, edge_index, edge_type, edge_table, Win, b_in, Ws1, as1, ad1,
           We1, ae1, b1, Ws2, as2, ad2, We2, ae2, b2, Wout, bout):
    src = edge_index[0]
    dst = edge_index[1]
    pad = E_PAD - E
    j = jnp.arange(pad, dtype=jnp.int32)
    src_p = jnp.concatenate([src, j % N])
    dst_p = jnp.concatenate([dst, N + (j % (2 * LANES))])
    et_p = jnp.concatenate([edge_type, jnp.zeros((pad,), jnp.int32)])
    etab_p = jnp.pad(edge_table, ((0, 128 - edge_table.shape[0]), (0, 0)))

    xs1, asrc1, adst1, ta1 = _tc_stage1(
        x, Win, b_in.reshape(1, D), Ws1, as1.reshape(D, 1), ad1.reshape(D, 1),
        etab_p, We1, ae1.reshape(D, 1))
    agg1, den1 = _gat_sc_layer(xs1, asrc1.reshape(N), adst1.reshape(N),
                               ta1.reshape(128), src_p, dst_p, et_p)
    xs2, asrc2, adst2, ta2 = _tc_stage2(
        agg1[0, :N], agg1[1, :N], den1[0, :N, None], den1[1, :N, None],
        b1.reshape(1, D), Ws2, as2.reshape(D, 1), ad2.reshape(D, 1),
        etab_p, We2, ae2.reshape(D, 1))
    agg2, den2 = _gat_sc_layer(xs2, asrc2.reshape(N), adst2.reshape(N),
                               ta2.reshape(128), src_p, dst_p, et_p)
    out = _tc_stage3(agg2[0, :N], agg2[1, :N], den2[0, :N, None],
                     den2[1, :N, None], b2.reshape(1, D), Wout,
                     bout.reshape(1, 1))
    return out


# speedup vs baseline: 35.2667x; 1.4730x over previous
import functools
import jax
import jax.numpy as jnp
from jax import lax
from jax.experimental import pallas as pl
from jax.experimental.pallas import tpu as pltpu
from jax.experimental.pallas import tpu_sc as plsc

N = 10000
D = 128
E = 320000
NC, NS, LANES = 2, 16, 16
NW = NC * NS
C = 96
CHUNKS = 105
EW = CHUNKS * C
E_PAD = EW * NW
N_PAD = 10240
RPT = N_PAD // NS
EPS = 1e-16


def _bcast_lane(v, lane):
    idx = jnp.full((LANES,), lane, dtype=jnp.int32)
    dn = lax.GatherDimensionNumbers(
        offset_dims=(), collapsed_slice_dims=(0,), start_index_map=(0,))
    return lax.gather(v, idx[:, None], dn, (1,),
                      mode=lax.GatherScatterMode.PROMISE_IN_BOUNDS)


def _gat_sc_body(xs_hbm, asrc_hbm, adst_hbm, ta_hbm, idx_hbm,
                 agg_out, den_out,
                 asrc_v, adst_v, ta_v, idx_a, idx_b, ex_a, ex_b,
                 rows_a, rows_b, zden_v, agg_s, den_s,
                 sem_ga, sem_gb, sem_sa, sem_sb):
    c = lax.axis_index("c")
    s = lax.axis_index("s")
    w = s * NC + c
    zero = jnp.zeros((LANES,), jnp.float32)

    pltpu.sync_copy(asrc_hbm, asrc_v)
    pltpu.sync_copy(adst_hbm, adst_v.at[pl.ds(0, N)])
    pltpu.sync_copy(ta_hbm, ta_v)
    adst_v[pl.ds(N, LANES)] = zero
    adst_v[pl.ds(N + LANES, LANES)] = zero

    @pl.loop(0, C)
    def _zr(r):
        for cv in range(8):
            rows_a[r, pl.ds(cv * LANES, LANES)] = zero

    @pl.loop(0, RPT // LANES)
    def _zd(i):
        zden_v[pl.ds(i * LANES, LANES)] = zero

    row0 = s * RPT
    for k in range(RPT // 64):
        pltpu.sync_copy(rows_a.at[pl.ds(0, 64)],
                        agg_s.at[pl.ds(row0 + k * 64, 64)])
    pltpu.sync_copy(zden_v, den_s.at[pl.ds(row0, RPT)])
    plsc.subcore_barrier()

    def _load_idx(k, idx):
        pltpu.sync_copy(idx_hbm.at[w, k], idx)

    def _gather(idx, rows, sem):
        pltpu.async_copy(xs_hbm.at[idx.at[0]], rows, sem)

    def _process(k, idx, ex_v, rows, sem_g, sem_s, idx_o, rows_o,
                 sem_go, sem_so):
        pltpu.make_async_copy(xs_hbm.at[idx.at[0]], rows, sem_g).wait()

        @pl.when(k > 0)
        def _():
            pltpu.make_async_copy(rows_o, agg_s.at[idx.at[1]], sem_so).wait()

        @pl.when(k + 1 < CHUNKS)
        def _():
            _load_idx(k + 1, idx_o)
            _gather(idx_o, rows_o, sem_go)

        for j in range(C // LANES):
            sl = pl.ds(j * LANES, LANES)
            a = (plsc.load_gather(asrc_v, [idx[0, sl]])
                 + plsc.load_gather(adst_v, [idx[1, sl]])
                 + plsc.load_gather(ta_v, [idx[2, sl]]))
            a = jnp.maximum(a, 0.2 * a)
            ex_v[sl] = jnp.exp(a)
        pltpu.sync_copy(ex_v, den_s.at[idx.at[1]], add=True)

        @pl.loop(0, C // LANES)
        def _scale(j):
            exr = ex_v[pl.ds(j * LANES, LANES)]
            for lane in range(LANES):
                b = _bcast_lane(exr, lane)
                e = j * LANES + lane
                for cv in range(8):
                    sl2 = pl.ds(cv * LANES, LANES)
                    rows[e, sl2] = rows[e, sl2] * b

        pltpu.async_copy(rows, agg_s.at[idx.at[1]], sem_s, add=True)

    _load_idx(0, idx_a)
    _gather(idx_a, rows_a, sem_ga)

    @pl.loop(0, CHUNKS, step=2)
    def _chunk(k):
        _process(k, idx_a, ex_a, rows_a, sem_ga, sem_sa,
                 idx_b, rows_b, sem_gb, sem_sb)

        @pl.when(k + 1 < CHUNKS)
        def _():
            _process(k + 1, idx_b, ex_b, rows_b, sem_gb, sem_sb,
                     idx_a, rows_a, sem_ga, sem_sa)

    pltpu.make_async_copy(rows_a, agg_s.at[idx_a.at[1]], sem_sa).wait()
    plsc.subcore_barrier()
    pltpu.sync_copy(agg_s.at[pl.ds(row0, RPT)],
                    agg_out.at[c, pl.ds(row0, RPT)])
    pltpu.sync_copy(den_s.at[pl.ds(row0, RPT)],
                    den_out.at[c, pl.ds(row0, RPT)])


def _gat_sc_layer(xs, asrc, adst, ta, idx_packed):
    mesh = plsc.VectorSubcoreMesh(core_axis_name="c", subcore_axis_name="s",
                                  num_cores=NC, num_subcores=NS)
    f = pl.kernel(
        _gat_sc_body,
        out_type=(jax.ShapeDtypeStruct((NC, N_PAD, D), jnp.float32),
                  jax.ShapeDtypeStruct((NC, N_PAD), jnp.float32)),
        mesh=mesh,
        scratch_types=[
            pltpu.VMEM((N,), jnp.float32),
            pltpu.VMEM((N + 2 * LANES,), jnp.float32),
            pltpu.VMEM((128,), jnp.float32),
            pltpu.VMEM((3, C), jnp.int32),
            pltpu.VMEM((3, C), jnp.int32),
            pltpu.VMEM((C,), jnp.float32),
            pltpu.VMEM((C,), jnp.float32),
            pltpu.VMEM((C, D), jnp.float32),
            pltpu.VMEM((C, D), jnp.float32),
            pltpu.VMEM((RPT,), jnp.float32),
            pltpu.VMEM_SHARED((N_PAD, D), jnp.float32),
            pltpu.VMEM_SHARED((N_PAD,), jnp.float32),
            pltpu.SemaphoreType.DMA,
            pltpu.SemaphoreType.DMA,
            pltpu.SemaphoreType.DMA,
            pltpu.SemaphoreType.DMA,
        ],
        compiler_params=pltpu.CompilerParams(needs_layout_passes=False),
        name="gat_edge_aggregate",
    )
    return f(xs, asrc, adst, ta, idx_packed)


ROWS_BLK = 400
GRID = N // ROWS_BLK


def _stage1_body(x_ref, win_ref, bin_ref, ws_ref, as_ref, ad_ref,
                 etab_ref, we_ref, ae_ref,
                 xs_ref, asrc_ref, adst_ref, ta_ref):
    h = jnp.dot(x_ref[...], win_ref[...],
                preferred_element_type=jnp.float32) + bin_ref[...]
    xs = jnp.dot(h, ws_ref[...], preferred_element_type=jnp.float32)
    xs_ref[...] = xs
    asrc_ref[...] = jnp.dot(xs, as_ref[...], preferred_element_type=jnp.float32)
    adst_ref[...] = jnp.dot(xs, ad_ref[...], preferred_element_type=jnp.float32)
    ee = jnp.dot(etab_ref[...], we_ref[...], preferred_element_type=jnp.float32)
    ta_ref[...] = jnp.dot(ee, ae_ref[...], preferred_element_type=jnp.float32)


def _stage2_body(p0_ref, p1_ref, d0_ref, d1_ref, bprev_ref, ws_ref, as_ref,
                 ad_ref, etab_ref, we_ref, ae_ref,
                 xs_ref, asrc_ref, adst_ref, ta_ref):
    agg = p0_ref[...] + p1_ref[...]
    den = d0_ref[...] + d1_ref[...] + EPS
    h = jnp.maximum(agg / den + bprev_ref[...], 0.0)
    xs = jnp.dot(h, ws_ref[...], preferred_element_type=jnp.float32)
    xs_ref[...] = xs
    asrc_ref[...] = jnp.dot(xs, as_ref[...], preferred_element_type=jnp.float32)
    adst_ref[...] = jnp.dot(xs, ad_ref[...], preferred_element_type=jnp.float32)
    ee = jnp.dot(etab_ref[...], we_ref[...], preferred_element_type=jnp.float32)
    ta_ref[...] = jnp.dot(ee, ae_ref[...], preferred_element_type=jnp.float32)


def _stage3_body(p0_ref, p1_ref, d0_ref, d1_ref, b2_ref, wout_ref, bout_ref,
                 out_ref):
    agg = p0_ref[...] + p1_ref[...]
    den = d0_ref[...] + d1_ref[...] + EPS
    h = jnp.maximum(agg / den + b2_ref[...], 0.0)
    out_ref[...] = jnp.dot(h, wout_ref[...],
                           preferred_element_type=jnp.float32) + bout_ref[...]


def _row_spec(blk):
    return pl.BlockSpec(blk, lambda i: (0,) * len(blk))


def _blk_spec(blk):
    return pl.BlockSpec(blk, lambda i: (i,) + (0,) * (len(blk) - 1))


def _tc_stage1(x, win, b_in, ws, a_s, a_d, etab, we, a_e):
    return pl.pallas_call(
        _stage1_body,
        grid=(GRID,),
        in_specs=[
            _blk_spec((ROWS_BLK, D)),
            _row_spec((D, D)), _row_spec((1, D)), _row_spec((D, D)),
            _row_spec((D, 1)), _row_spec((D, 1)),
            _row_spec((128, 16)), _row_spec((16, D)), _row_spec((D, 1)),
        ],
        out_specs=[
            _blk_spec((ROWS_BLK, D)), _blk_spec((ROWS_BLK, 1)),
            _blk_spec((ROWS_BLK, 1)), _row_spec((128, 1)),
        ],
        out_shape=[
            jax.ShapeDtypeStruct((N, D), jnp.float32),
            jax.ShapeDtypeStruct((N, 1), jnp.float32),
            jax.ShapeDtypeStruct((N, 1), jnp.float32),
            jax.ShapeDtypeStruct((128, 1), jnp.float32),
        ],
    )(x, win, b_in, ws, a_s, a_d, etab, we, a_e)


def _tc_stage2(p0, p1, d0, d1, bprev, ws, a_s, a_d, etab, we, a_e):
    return pl.pallas_call(
        _stage2_body,
        grid=(GRID,),
        in_specs=[
            _blk_spec((ROWS_BLK, D)), _blk_spec((ROWS_BLK, D)),
            _blk_spec((ROWS_BLK, 1)), _blk_spec((ROWS_BLK, 1)),
            _row_spec((1, D)), _row_spec((D, D)),
            _row_spec((D, 1)), _row_spec((D, 1)),
            _row_spec((128, 16)), _row_spec((16, D)), _row_spec((D, 1)),
        ],
        out_specs=[
            _blk_spec((ROWS_BLK, D)), _blk_spec((ROWS_BLK, 1)),
            _blk_spec((ROWS_BLK, 1)), _row_spec((128, 1)),
        ],
        out_shape=[
            jax.ShapeDtypeStruct((N, D), jnp.float32),
            jax.ShapeDtypeStruct((N, 1), jnp.float32),
            jax.ShapeDtypeStruct((N, 1), jnp.float32),
            jax.ShapeDtypeStruct((128, 1), jnp.float32),
        ],
    )(p0, p1, d0, d1, bprev, ws, a_s, a_d, etab, we, a_e)


def _tc_stage3(p0, p1, d0, d1, b2, wout, bout):
    return pl.pallas_call(
        _stage3_body,
        grid=(GRID,),
        in_specs=[
            _blk_spec((ROWS_BLK, D)), _blk_spec((ROWS_BLK, D)),
            _blk_spec((ROWS_BLK, 1)), _blk_spec((ROWS_BLK, 1)),
            _row_spec((1, D)), _row_spec((D, 1)), _row_spec((1, 1)),
        ],
        out_specs=_blk_spec((ROWS_BLK, 1)),
        out_shape=jax.ShapeDtypeStruct((N, 1), jnp.float32),
    )(p0, p1, d0, d1, b2, wout, bout)


@jax.jit
def kernel(x, edge_index, edge_type, edge_table, Win, b_in, Ws1, as1, ad1,
           We1, ae1, b1, Ws2, as2, ad2, We2, ae2, b2, Wout, bout):
    src = edge_index[0]
    dst = edge_index[1]
    pad = E_PAD - E
    j = jnp.arange(pad, dtype=jnp.int32)
    src_p = jnp.concatenate([src, j % N]).reshape(NW, CHUNKS, C)
    dst_p = jnp.concatenate([dst, N + (j % (2 * LANES))]).reshape(NW, CHUNKS, C)
    et_p = jnp.concatenate([edge_type, jnp.zeros((pad,), jnp.int32)]
                           ).reshape(NW, CHUNKS, C)
    idx_packed = jnp.stack([src_p, dst_p, et_p], axis=2)
    etab_p = jnp.pad(edge_table, ((0, 128 - edge_table.shape[0]), (0, 0)))

    xs1, asrc1, adst1, ta1 = _tc_stage1(
        x, Win, b_in.reshape(1, D), Ws1, as1.reshape(D, 1), ad1.reshape(D, 1),
        etab_p, We1, ae1.reshape(D, 1))
    agg1, den1 = _gat_sc_layer(xs1, asrc1.reshape(N), adst1.reshape(N),
                               ta1.reshape(128), idx_packed)
    xs2, asrc2, adst2, ta2 = _tc_stage2(
        agg1[0, :N], agg1[1, :N], den1[0, :N, None], den1[1, :N, None],
        b1.reshape(1, D), Ws2, as2.reshape(D, 1), ad2.reshape(D, 1),
        etab_p, We2, ae2.reshape(D, 1))
    agg2, den2 = _gat_sc_layer(xs2, asrc2.reshape(N), adst2.reshape(N),
                               ta2.reshape(128), idx_packed)
    out = _tc_stage3(agg2[0, :N], agg2[1, :N], den2[0, :N, None],
                     den2[1, :N, None], b2.reshape(1, D), Wout,
                     bout.reshape(1, 1))
    return out
